# Initial kernel scaffold; baseline (speedup 1.0000x reference)
import functools
import jax
import jax.numpy as jnp
from jax import lax
from jax.experimental import pallas as pl
from jax.experimental.pallas import tpu as pltpu, tpu_sc as plsc

_S = 10000
_NW = 32


def kernel(features, agg_indices, fc_w, fc_b):
    N, D = features.shape
    h = features

    mesh = plsc.VectorSubcoreMesh(core_axis_name="c", subcore_axis_name="s")

    @functools.partial(
        pl.kernel,
        out_type=(
            jax.ShapeDtypeStruct((N, 2 * D), jnp.float32),
            jax.ShapeDtypeStruct((_S, D), jnp.float32),
        ),
        mesh=mesh,
        scratch_types=[
            pltpu.VMEM((256, 128), jnp.float32),
            pltpu.VMEM((272,), jnp.int32),
            pltpu.SemaphoreType.DMA,
        ],
    )
    def body(h_hbm, idx_hbm, out_hbm, agg_hbm, hbuf, ibuf, sem):
        wid = lax.axis_index("s") * 2 + lax.axis_index("c")
        base = wid * (N // _NW)
        pltpu.sync_copy(idx_hbm.at[pl.ds(base, 256)], ibuf.at[pl.ds(0, 256)])
        pltpu.sync_copy(h_hbm.at[pl.ds(base, 256), :], hbuf)

        def loop(i, carry):
            acc, cnt = carry
            v16 = ibuf[pl.ds(i * 16, 16)]
            m = jnp.max(v16)
            lane = jnp.max(jnp.where(lax.iota(jnp.int32, 16) == (i % 16), v16, -(2**31 - 1)))
            b16 = jnp.full((16,), lane, jnp.int32)
            row = hbuf[i, pl.ds(0, 16)]
            acc = jnp.maximum(acc, row)
            colv = lax.iota(jnp.int32, 16)
            g = plsc.load_gather(hbuf, [b16 & 0xFF, colv])
            plsc.store_scatter(hbuf, [b16 & 0xFF, colv], jnp.maximum(g, acc))
            @pl.when(m > 3)
            def _():
                hbuf[0, pl.ds(0, 16)] = acc

            hbuf[i + 1, pl.ds(16, 16)] = acc
            return acc, cnt + m

        acc0 = jnp.zeros((16,), jnp.float32)
        acc, cnt = lax.fori_loop(0, 8, loop, (acc0, jnp.int32(0)))

        @pl.when(cnt > 0)
        def _():
            pltpu.sync_copy(hbuf.at[pl.ds(0, 64), :], out_hbm.at[pl.ds(base, 64), pl.ds(D, D)])

        pltpu.sync_copy(hbuf.at[pl.ds(0, 64), :], out_hbm.at[pl.ds(base + 64, 64), pl.ds(0, D)])
        pltpu.sync_copy(hbuf.at[pl.ds(0, 1), :], agg_hbm.at[pl.ds(wid, 1), :])

    out, agg = body(h, agg_indices)
    return out, agg


# speedup vs baseline: 1.6221x; 1.6221x over previous
import functools

import jax
import jax.numpy as jnp
from jax import lax
from jax.experimental import pallas as pl
from jax.experimental.pallas import tpu as pltpu
from jax.experimental.pallas import tpu_sc as plsc

_NUM_SEGMENTS = 10000
_LANES = 16


def _worker_id(nc):
    return lax.axis_index("s") * nc + lax.axis_index("c")


def _linear_leaky(features, fc_wT, fc_b2d, blk):
    n, d_in = features.shape
    d_out = fc_wT.shape[1]

    def body(x_ref, w_ref, b_ref, o_ref):
        y = jnp.dot(x_ref[...], w_ref[...], preferred_element_type=jnp.float32)
        y = y + b_ref[...]
        o_ref[...] = jnp.where(y >= 0, y, 0.01 * y)

    return pl.pallas_call(
        body,
        grid=(n // blk,),
        in_specs=[
            pl.BlockSpec((blk, d_in), lambda i: (i, 0)),
            pl.BlockSpec((d_in, d_out), lambda i: (0, 0)),
            pl.BlockSpec((1, d_out), lambda i: (0, 0)),
        ],
        out_specs=pl.BlockSpec((blk, d_out), lambda i: (i, 0)),
        out_shape=jax.ShapeDtypeStruct((n, d_out), jnp.float32),
        compiler_params=pltpu.CompilerParams(
            dimension_semantics=("arbitrary",),
        ),
    )(features, fc_wT, fc_b2d)


def _seg_max_concat(h, idx_padded, n, d, num_segments, nw, chunk):
    p = n // nw

    mesh = plsc.VectorSubcoreMesh(core_axis_name="c", subcore_axis_name="s")

    @functools.partial(
        pl.kernel,
        out_type=(
            jax.ShapeDtypeStruct((n, 2 * d), jnp.float32),
            jax.ShapeDtypeStruct((num_segments, d), jnp.float32),
        ),
        mesh=mesh,
        compiler_params=pltpu.CompilerParams(needs_layout_passes=False,
                                             use_tc_tiling_on_sc=False),
        scratch_types=[
            pltpu.VMEM((chunk, d), jnp.float32),
            pltpu.VMEM((chunk + 32,), jnp.int32),
            pltpu.VMEM((128, d), jnp.float32),
            pltpu.VMEM((64, d), jnp.float32),
        ],
    )
    def body(h_hbm, idx_hbm, out_hbm, agg_hbm, hbuf, ibuf, rep, zbuf):
        info = plsc.get_sparse_core_info()
        nc = info.num_cores
        wid = _worker_id(nc)
        _worker_body(wid, n, d, num_segments, nw, p, chunk,
                     h_hbm, idx_hbm, out_hbm, agg_hbm, hbuf, ibuf, rep, zbuf)

    return body(h, idx_padded)


def _worker_body(wid, n, d, num_segments, nw, p, chunk,
                 h_hbm, idx_hbm, out_hbm, agg_hbm, hbuf, ibuf, rep, zbuf):
    if True:
        rw0 = wid * p
        rw1 = rw0 + p

        zvec = jnp.zeros((_LANES,), jnp.float32)

        def zrow(i, _):
            for j in range(d // _LANES):
                zbuf[i, pl.ds(_LANES * j, _LANES)] = zvec
            return 0

        lax.fori_loop(0, 64, zrow, 0)

        @pl.when(wid > 0)
        def _():
            pltpu.sync_copy(
                idx_hbm.at[pl.ds(pl.multiple_of(rw0 - _LANES, 8), _LANES)],
                ibuf.at[pl.ds(0, _LANES)])

        lo = jnp.where(wid > 0, ibuf[pl.ds(0, _LANES)][_LANES - 1] + 1, 0)
        pltpu.sync_copy(
            idx_hbm.at[pl.ds(pl.multiple_of(rw1 - _LANES, 8), _LANES)],
            ibuf.at[pl.ds(0, _LANES)])
        hi = jnp.where(wid < nw - 1, ibuf[pl.ds(0, _LANES)][_LANES - 1] + 1,
                       num_segments)

        def blk_write(copy_fn, start, count, maxblk):
            nfull = count // maxblk

            def df(i, _):
                copy_fn(start + i * maxblk, maxblk)
                return 0

            lax.fori_loop(0, nfull, df, 0)
            sz = maxblk // 2
            while sz >= 1:
                pos = start + count - (count % (2 * sz))

                def mk(pos=pos, sz=sz):
                    @pl.when((count & sz) != 0)
                    def _():
                        copy_fn(pos, sz)

                mk()
                sz //= 2

        def flush(cur, a, b, zfrom, accs):
            def zcopy(pos, sz):
                pltpu.sync_copy(zbuf.at[pl.ds(0, sz), :],
                                agg_hbm.at[pl.ds(pos, sz), :])

            blk_write(zcopy, zfrom, cur - zfrom, 64)

            nrows = b - a
            fill = jnp.minimum(nrows, 128)

            def frow(i, _):
                for j in range(d // _LANES):
                    rep[i, pl.ds(_LANES * j, _LANES)] = accs[j]
                return 0

            lax.fori_loop(0, fill, frow, 0)

            pltpu.sync_copy(rep.at[pl.ds(0, 1), :], agg_hbm.at[pl.ds(cur, 1), :])

            def ocopy(pos, sz):
                pltpu.sync_copy(rep.at[pl.ds(0, sz), :],
                                out_hbm.at[pl.ds(pos, sz), pl.ds(d, d)])

            blk_write(ocopy, a, nrows, 128)

        neg = jnp.full((_LANES,), -3.0e38, jnp.float32)

        def process_chunk(r, st, copy_left):
            r_al = pl.multiple_of(r - lax.rem(r, 8), 8)
            off = r - r_al
            pltpu.sync_copy(idx_hbm.at[pl.ds(r_al, chunk + 8)],
                            ibuf.at[pl.ds(0, chunk + 8)])
            pltpu.sync_copy(h_hbm.at[pl.ds(r, chunk), :], hbuf)
            if copy_left:
                pltpu.sync_copy(hbuf, out_hbm.at[pl.ds(r, chunk), pl.ds(0, d)])

            def row_body(k, st):
                s = ibuf[pl.ds(off + k, _LANES)][0]
                rowv = tuple(hbuf[k, pl.ds(_LANES * j, _LANES)]
                             for j in range(d // _LANES))

                def active_fn(st):
                    cur, a, b, zfrom = st[:4]
                    accs = st[4:]

                    def same_fn(_):
                        naccs = tuple(jnp.maximum(accs[j], rowv[j])
                                      for j in range(d // _LANES))
                        return (cur, a, r + k + 1, zfrom) + naccs

                    def diff_fn(_):
                        @pl.when(cur >= 0)
                        def _():
                            flush(cur, a, b, zfrom, accs)

                        nzfrom = jnp.where(cur >= 0, cur + 1, zfrom)
                        return (s, r + k, r + k + 1, nzfrom) + rowv

                    return lax.cond(s == cur, same_fn, diff_fn, 0)

                return lax.cond((s >= lo) & (s < hi), active_fn,
                                lambda st: st, st)

            return lax.fori_loop(0, chunk, row_body, st)

        st0 = (jnp.int32(-1), rw0, rw0, lo) + tuple(neg for _ in range(d // _LANES))

        def main_chunk(i, st):
            return process_chunk(rw0 + i * chunk, st, True)

        st = lax.fori_loop(0, p // chunk, main_chunk, st0)

        def over_cond(carry):
            r, stopped = carry[0], carry[1]
            return (r < n) & jnp.logical_not(stopped)

        def over_body(carry):
            r = carry[0]
            st = carry[2:]
            r_al = r - lax.rem(r, 8)
            off = r - r_al
            st = process_chunk(r, st, False)
            last = ibuf[pl.ds(off + chunk - _LANES, _LANES)][_LANES - 1]
            return (r + chunk, last >= hi) + st

        carry = (rw1, jnp.bool_(False)) + st
        carry = lax.while_loop(over_cond, over_body, carry)
        st = carry[2:]

        cur, a, b, zfrom = st[:4]
        accs = st[4:]

        @pl.when(cur >= 0)
        def _():
            flush(cur, a, b, zfrom, accs)

        ztail = jnp.where(cur >= 0, cur + 1, zfrom)

        def zcopy2(pos, sz):
            pltpu.sync_copy(zbuf.at[pl.ds(0, sz), :],
                            agg_hbm.at[pl.ds(pos, sz), :])

        blk_write(zcopy2, ztail, hi - ztail, 64)


def kernel(features, agg_indices, fc_w, fc_b):
    n, d_in = features.shape
    d_out = fc_w.shape[0]

    h = _linear_leaky(features, fc_w.T, fc_b.reshape(1, d_out), blk=640)

    idx32 = agg_indices.astype(jnp.int32)
    idx_padded = jnp.concatenate(
        [idx32, jnp.full((_LANES,), _NUM_SEGMENTS, jnp.int32)])

    out, agg = _seg_max_concat(h, idx_padded, n, d_out, _NUM_SEGMENTS,
                               nw=32, chunk=200)
    return out, agg


# speedup vs baseline: 1.8919x; 1.1663x over previous
import functools

import jax
import jax.numpy as jnp
from jax import lax
from jax.experimental import pallas as pl
from jax.experimental.pallas import tpu as pltpu
from jax.experimental.pallas import tpu_sc as plsc

_NUM_SEGMENTS = 10000
_LANES = 16


def _worker_id(nc):
    return lax.axis_index("s") * nc + lax.axis_index("c")


def _linear_leaky(features, fc_wT, fc_b2d, blk):
    n, d_in = features.shape
    d_out = fc_wT.shape[1]

    def body(x_ref, w_ref, b_ref, o_ref):
        y = jnp.dot(x_ref[...], w_ref[...], preferred_element_type=jnp.float32)
        y = y + b_ref[...]
        o_ref[...] = jnp.where(y >= 0, y, 0.01 * y)

    return pl.pallas_call(
        body,
        grid=(n // blk,),
        in_specs=[
            pl.BlockSpec((blk, d_in), lambda i: (i, 0)),
            pl.BlockSpec((d_in, d_out), lambda i: (0, 0)),
            pl.BlockSpec((1, d_out), lambda i: (0, 0)),
        ],
        out_specs=pl.BlockSpec((blk, d_out), lambda i: (i, 0)),
        out_shape=jax.ShapeDtypeStruct((n, d_out), jnp.float32),
        compiler_params=pltpu.CompilerParams(
            dimension_semantics=("arbitrary",),
        ),
    )(features, fc_wT, fc_b2d)


def _seg_max_concat(h, idx_padded, n, d, num_segments, nw, chunk):
    p = n // nw

    mesh = plsc.VectorSubcoreMesh(core_axis_name="c", subcore_axis_name="s")

    @functools.partial(
        pl.kernel,
        out_type=(
            jax.ShapeDtypeStruct((n, d), jnp.float32),
            jax.ShapeDtypeStruct((num_segments, d), jnp.float32),
        ),
        mesh=mesh,
        compiler_params=pltpu.CompilerParams(needs_layout_passes=False,
                                             use_tc_tiling_on_sc=False),
        scratch_types=[
            pltpu.VMEM((chunk, d), jnp.float32),
            pltpu.VMEM((chunk + 32,), jnp.int32),
            pltpu.VMEM((128, d), jnp.float32),
            pltpu.VMEM((64, d), jnp.float32),
        ],
    )
    def body(h_hbm, idx_hbm, out_hbm, agg_hbm, hbuf, ibuf, rep, zbuf):
        info = plsc.get_sparse_core_info()
        nc = info.num_cores
        wid = _worker_id(nc)
        _worker_body(wid, n, d, num_segments, nw, p, chunk,
                     h_hbm, idx_hbm, out_hbm, agg_hbm, hbuf, ibuf, rep, zbuf)

    return body(h, idx_padded)


def _worker_body(wid, n, d, num_segments, nw, p, chunk,
                 h_hbm, idx_hbm, out_hbm, agg_hbm, hbuf, ibuf, rep, zbuf):
    if True:
        rw0 = wid * p
        rw1 = rw0 + p

        zvec = jnp.zeros((_LANES,), jnp.float32)

        def zrow(i, _):
            for j in range(d // _LANES):
                zbuf[i, pl.ds(_LANES * j, _LANES)] = zvec
            return 0

        lax.fori_loop(0, 64, zrow, 0)

        @pl.when(wid > 0)
        def _():
            pltpu.sync_copy(
                idx_hbm.at[pl.ds(pl.multiple_of(rw0 - _LANES, 8), _LANES)],
                ibuf.at[pl.ds(0, _LANES)])

        lo = jnp.where(wid > 0, ibuf[pl.ds(0, _LANES)][_LANES - 1] + 1, 0)
        pltpu.sync_copy(
            idx_hbm.at[pl.ds(pl.multiple_of(rw1 - _LANES, 8), _LANES)],
            ibuf.at[pl.ds(0, _LANES)])
        hi = jnp.where(wid < nw - 1, ibuf[pl.ds(0, _LANES)][_LANES - 1] + 1,
                       num_segments)

        def blk_write(copy_fn, start, count, maxblk):
            nfull = count // maxblk

            def df(i, _):
                copy_fn(start + i * maxblk, maxblk)
                return 0

            lax.fori_loop(0, nfull, df, 0)
            sz = maxblk // 2
            while sz >= 1:
                pos = start + count - (count % (2 * sz))

                def mk(pos=pos, sz=sz):
                    @pl.when((count & sz) != 0)
                    def _():
                        copy_fn(pos, sz)

                mk()
                sz //= 2

        def flush(cur, a, b, zfrom, accs):
            def zcopy(pos, sz):
                pltpu.sync_copy(zbuf.at[pl.ds(0, sz), :],
                                agg_hbm.at[pl.ds(pos, sz), :])

            blk_write(zcopy, zfrom, cur - zfrom, 64)

            nrows = b - a
            fill = jnp.minimum(nrows, 128)

            def frow(i, _):
                for j in range(d // _LANES):
                    rep[i, pl.ds(_LANES * j, _LANES)] = accs[j]
                return 0

            lax.fori_loop(0, fill, frow, 0)

            pltpu.sync_copy(rep.at[pl.ds(0, 1), :], agg_hbm.at[pl.ds(cur, 1), :])

            def ocopy(pos, sz):
                pltpu.sync_copy(rep.at[pl.ds(0, sz), :],
                                out_hbm.at[pl.ds(pos, sz), :])

            blk_write(ocopy, a, nrows, 128)

        neg = jnp.full((_LANES,), -3.0e38, jnp.float32)

        def process_chunk(r, st):
            r_al = pl.multiple_of(r - lax.rem(r, 8), 8)
            off = r - r_al
            pltpu.sync_copy(idx_hbm.at[pl.ds(r_al, chunk + 8)],
                            ibuf.at[pl.ds(0, chunk + 8)])
            pltpu.sync_copy(h_hbm.at[pl.ds(r, chunk), :], hbuf)

            def row_body(k, st):
                s = ibuf[pl.ds(off + k, _LANES)][0]
                rowv = tuple(hbuf[k, pl.ds(_LANES * j, _LANES)]
                             for j in range(d // _LANES))

                def active_fn(st):
                    cur, a, b, zfrom = st[:4]
                    accs = st[4:]

                    def same_fn(_):
                        naccs = tuple(jnp.maximum(accs[j], rowv[j])
                                      for j in range(d // _LANES))
                        return (cur, a, r + k + 1, zfrom) + naccs

                    def diff_fn(_):
                        @pl.when(cur >= 0)
                        def _():
                            flush(cur, a, b, zfrom, accs)

                        nzfrom = jnp.where(cur >= 0, cur + 1, zfrom)
                        return (s, r + k, r + k + 1, nzfrom) + rowv

                    return lax.cond(s == cur, same_fn, diff_fn, 0)

                return lax.cond((s >= lo) & (s < hi), active_fn,
                                lambda st: st, st)

            return lax.fori_loop(0, chunk, row_body, st)

        st0 = (jnp.int32(-1), rw0, rw0, lo) + tuple(neg for _ in range(d // _LANES))

        def main_chunk(i, st):
            return process_chunk(rw0 + i * chunk, st)

        st = lax.fori_loop(0, p // chunk, main_chunk, st0)

        def over_cond(carry):
            r, stopped = carry[0], carry[1]
            return (r < n) & jnp.logical_not(stopped)

        def over_body(carry):
            r = carry[0]
            st = carry[2:]
            r_al = r - lax.rem(r, 8)
            off = r - r_al
            st = process_chunk(r, st)
            last = ibuf[pl.ds(off + chunk - _LANES, _LANES)][_LANES - 1]
            return (r + chunk, last >= hi) + st

        carry = (rw1, jnp.bool_(False)) + st
        carry = lax.while_loop(over_cond, over_body, carry)
        st = carry[2:]

        cur, a, b, zfrom = st[:4]
        accs = st[4:]

        @pl.when(cur >= 0)
        def _():
            flush(cur, a, b, zfrom, accs)

        ztail = jnp.where(cur >= 0, cur + 1, zfrom)

        def zcopy2(pos, sz):
            pltpu.sync_copy(zbuf.at[pl.ds(0, sz), :],
                            agg_hbm.at[pl.ds(pos, sz), :])

        blk_write(zcopy2, ztail, hi - ztail, 64)


def kernel(features, agg_indices, fc_w, fc_b):
    n, d_in = features.shape
    d_out = fc_w.shape[0]

    h = _linear_leaky(features, fc_w.T, fc_b.reshape(1, d_out), blk=640)

    idx32 = agg_indices.astype(jnp.int32)
    idx_padded = jnp.concatenate(
        [idx32, jnp.full((_LANES,), _NUM_SEGMENTS, jnp.int32)])

    out_right, agg = _seg_max_concat(h, idx_padded, n, d_out, _NUM_SEGMENTS,
                                     nw=32, chunk=200)
    out = jnp.concatenate([h, out_right], axis=-1)
    return out, agg


# speedup vs baseline: 2.3670x; 1.2511x over previous
import functools

import jax
import jax.numpy as jnp
from jax import lax
from jax.experimental import pallas as pl
from jax.experimental.pallas import tpu as pltpu
from jax.experimental.pallas import tpu_sc as plsc

_NUM_SEGMENTS = 10000
_LANES = 16
_AW = 64


def _worker_id(nc):
    return lax.axis_index("s") * nc + lax.axis_index("c")


def _linear_leaky(features, fc_wT, fc_b2d, blk):
    n, d_in = features.shape
    d_out = fc_wT.shape[1]

    def body(x_ref, w_ref, b_ref, o_ref):
        y = jnp.dot(x_ref[...], w_ref[...], preferred_element_type=jnp.float32)
        y = y + b_ref[...]
        o_ref[...] = jnp.where(y >= 0, y, 0.01 * y)

    return pl.pallas_call(
        body,
        grid=(n // blk,),
        in_specs=[
            pl.BlockSpec((blk, d_in), lambda i: (i, 0)),
            pl.BlockSpec((d_in, d_out), lambda i: (0, 0)),
            pl.BlockSpec((1, d_out), lambda i: (0, 0)),
        ],
        out_specs=pl.BlockSpec((blk, d_out), lambda i: (i, 0)),
        out_shape=jax.ShapeDtypeStruct((n, d_out), jnp.float32),
        compiler_params=pltpu.CompilerParams(
            dimension_semantics=("arbitrary",),
        ),
    )(features, fc_wT, fc_b2d)


def _seg_max_concat(h, idx_padded, n, d, num_segments, nw, chunk):
    p = n // nw

    mesh = plsc.VectorSubcoreMesh(core_axis_name="c", subcore_axis_name="s")

    @functools.partial(
        pl.kernel,
        out_type=(
            jax.ShapeDtypeStruct((n, d), jnp.float32),
            jax.ShapeDtypeStruct((num_segments, d), jnp.float32),
        ),
        mesh=mesh,
        compiler_params=pltpu.CompilerParams(needs_layout_passes=False,
                                             use_tc_tiling_on_sc=False),
        scratch_types=[
            pltpu.VMEM((chunk, d), jnp.float32),
            pltpu.VMEM((chunk + 32,), jnp.int32),
            pltpu.VMEM((chunk, d), jnp.float32),
            pltpu.VMEM((_AW, d), jnp.float32),
            pltpu.VMEM((64, d), jnp.float32),
        ],
    )
    def body(h_hbm, idx_hbm, out_hbm, agg_hbm, hbuf, ibuf, obuf, awbuf, rep):
        info = plsc.get_sparse_core_info()
        nc = info.num_cores
        wid = _worker_id(nc)
        _worker_body(wid, n, d, num_segments, nw, p, chunk,
                     h_hbm, idx_hbm, out_hbm, agg_hbm,
                     hbuf, ibuf, obuf, awbuf, rep)

    return body(h, idx_padded)


def _worker_body(wid, n, d, num_segments, nw, p, chunk,
                 h_hbm, idx_hbm, out_hbm, agg_hbm, hbuf, ibuf, obuf, awbuf, rep):
    nd = d // _LANES
    rw0 = wid * p
    rw1 = rw0 + p
    zvec = jnp.zeros((_LANES,), jnp.float32)
    big = jnp.int32(n + 2 * chunk)

    def memset_aw():
        def zrow(i, _):
            for j in range(nd):
                awbuf[i, pl.ds(_LANES * j, _LANES)] = zvec
            return 0

        lax.fori_loop(0, _AW, zrow, 0)

    memset_aw()

    @pl.when(wid > 0)
    def _():
        pltpu.sync_copy(
            idx_hbm.at[pl.ds(pl.multiple_of(rw0 - _LANES, 8), _LANES)],
            ibuf.at[pl.ds(0, _LANES)])

    lo = jnp.where(wid > 0, ibuf[pl.ds(0, _LANES)][_LANES - 1] + 1, 0)
    pltpu.sync_copy(
        idx_hbm.at[pl.ds(pl.multiple_of(rw1 - _LANES, 8), _LANES)],
        ibuf.at[pl.ds(0, _LANES)])
    hi = jnp.where(wid < nw - 1, ibuf[pl.ds(0, _LANES)][_LANES - 1] + 1,
                   num_segments)

    def blk_write(copy_fn, start, count, maxblk):
        nfull = count // maxblk

        def df(i, _):
            copy_fn(start + i * maxblk, maxblk)
            return 0

        lax.fori_loop(0, nfull, df, 0)
        sz = maxblk // 2
        while sz >= 1:
            pos = start + count - (count % (2 * sz))

            def mk(pos=pos, sz=sz):
                @pl.when((count & sz) != 0)
                def _():
                    copy_fn(pos, sz)

            mk()
            sz //= 2

    def aw_advance(cur, aw_lo):
        def cond(w):
            return cur >= w + _AW

        def adv(w):
            pltpu.sync_copy(awbuf, agg_hbm.at[pl.ds(w, _AW), :])
            memset_aw()
            return w + _AW

        return lax.while_loop(cond, adv, aw_lo)

    def rep_patch(a, cnt, accs):
        fill = jnp.minimum(cnt, 64)

        def frow(i, _):
            for j in range(nd):
                rep[i, pl.ds(_LANES * j, _LANES)] = accs[j]
            return 0

        lax.fori_loop(0, fill, frow, 0)

        def pcopy(pos, sz):
            pltpu.sync_copy(rep.at[pl.ds(0, sz), :],
                            out_hbm.at[pl.ds(pos, sz), :])

        blk_write(pcopy, a, cnt, 64)

    def flush_in_chunk(r, cur, a, b, aw_lo, accs):
        aw_lo = aw_advance(cur, aw_lo)
        for j in range(nd):
            awbuf[cur - aw_lo, pl.ds(_LANES * j, _LANES)] = accs[j]
        la = jnp.maximum(a, r)

        def srow(k, _):
            for j in range(nd):
                obuf[k, pl.ds(_LANES * j, _LANES)] = accs[j]
            return 0

        lax.fori_loop(la - r, b - r, srow, 0)

        @pl.when(a < r)
        def _():
            rep_patch(a, r - a, accs)

        return aw_lo

    neg = jnp.full((_LANES,), -3.0e38, jnp.float32)

    def process_chunk(r, st):
        r_al = pl.multiple_of(r - lax.rem(r, 8), 8)
        off = r - r_al
        pltpu.sync_copy(idx_hbm.at[pl.ds(r_al, chunk + 8)],
                        ibuf.at[pl.ds(0, chunk + 8)])
        pltpu.sync_copy(h_hbm.at[pl.ds(r, chunk), :], hbuf)

        def row_body(k, st):
            s = ibuf[pl.ds(off + k, _LANES)][0]
            rowv = tuple(hbuf[k, pl.ds(_LANES * j, _LANES)]
                         for j in range(nd))

            def active_fn(st):
                cur, a, b, fa, aw_lo = st[:5]
                accs = st[5:]

                def same_fn(_):
                    naccs = tuple(jnp.maximum(accs[j], rowv[j])
                                  for j in range(nd))
                    return (cur, a, r + k + 1, fa, aw_lo) + naccs

                def diff_fn(_):
                    naw = lax.cond(
                        cur >= 0,
                        lambda _: flush_in_chunk(r, cur, a, b, aw_lo, accs),
                        lambda _: aw_lo, 0)
                    nfa = jnp.where(fa < 0, r + k, fa)
                    return (s, r + k, r + k + 1, nfa, naw) + rowv

                return lax.cond(s == cur, same_fn, diff_fn, 0)

            return lax.cond((s >= lo) & (s < hi), active_fn,
                            lambda st: st, st)

        st = lax.fori_loop(0, chunk, row_body, st)

        cur, a, b, fa, aw_lo = st[:5]
        astart = jnp.maximum(r, jnp.where(fa < 0, big, fa))
        aend = jnp.minimum(b, r + chunk)
        cnt = aend - astart

        @pl.when(cnt > 0)
        def _():
            def ocopy(pos, sz):
                pltpu.sync_copy(obuf.at[pl.ds(pos - r, sz), :],
                                out_hbm.at[pl.ds(pos, sz), :])

            blk_write(ocopy, astart, cnt, min(128, chunk))

        return st

    st0 = (jnp.int32(-1), rw0, rw0, jnp.int32(-1), lo) + tuple(
        neg for _ in range(nd))

    def main_chunk(i, st):
        return process_chunk(rw0 + i * chunk, st)

    st = lax.fori_loop(0, p // chunk, main_chunk, st0)

    def over_cond(carry):
        r, stopped = carry[0], carry[1]
        return (r < n) & jnp.logical_not(stopped)

    def over_body(carry):
        r = carry[0]
        st = carry[2:]
        r_al = pl.multiple_of(r - lax.rem(r, 8), 8)
        off = r - r_al
        st = process_chunk(r, st)
        last = ibuf[pl.ds(off + chunk - _LANES, _LANES)][_LANES - 1]
        return (r + chunk, last >= hi) + st

    carry = (rw1, jnp.bool_(False)) + st
    carry = lax.while_loop(over_cond, over_body, carry)
    st = carry[2:]

    cur, a, b, fa, aw_lo = st[:5]
    accs = st[5:]

    def final_flush(aw_lo):
        naw = aw_advance(cur, aw_lo)
        for j in range(nd):
            awbuf[cur - naw, pl.ds(_LANES * j, _LANES)] = accs[j]
        rep_patch(a, b - a, accs)
        return naw

    aw_lo = lax.cond(cur >= 0, final_flush, lambda w: w, aw_lo)

    def tail_cond(w):
        return w + _AW <= hi

    def tail_adv(w):
        pltpu.sync_copy(awbuf, agg_hbm.at[pl.ds(w, _AW), :])
        memset_aw()
        return w + _AW

    aw_lo = lax.while_loop(tail_cond, tail_adv, aw_lo)

    def awcopy(pos, sz):
        pltpu.sync_copy(awbuf.at[pl.ds(pos - aw_lo, sz), :],
                        agg_hbm.at[pl.ds(pos, sz), :])

    blk_write(awcopy, aw_lo, hi - aw_lo, _AW // 2)


def kernel(features, agg_indices, fc_w, fc_b):
    n, d_in = features.shape
    d_out = fc_w.shape[0]

    h = _linear_leaky(features, fc_w.T, fc_b.reshape(1, d_out), blk=2000)

    idx32 = agg_indices.astype(jnp.int32)
    idx_padded = jnp.concatenate(
        [idx32, jnp.full((_LANES,), _NUM_SEGMENTS, jnp.int32)])

    out_right, agg = _seg_max_concat(h, idx_padded, n, d_out, _NUM_SEGMENTS,
                                     nw=32, chunk=200)
    out = jnp.concatenate([h, out_right], axis=-1)
    return out, agg


# speedup vs baseline: 2.6281x; 1.1103x over previous
import functools

import jax
import jax.numpy as jnp
from jax import lax
from jax.experimental import pallas as pl
from jax.experimental.pallas import tpu as pltpu
from jax.experimental.pallas import tpu_sc as plsc

_NUM_SEGMENTS = 10000
_LANES = 16
_AW = 64


def _worker_id(nc):
    return lax.axis_index("s") * nc + lax.axis_index("c")


def _linear_leaky(features, fc_wT, fc_b2d, blk):
    n, d_in = features.shape
    d_out = fc_wT.shape[1]

    def body(x_ref, w_ref, b_ref, o_ref):
        y = jnp.dot(x_ref[...], w_ref[...], preferred_element_type=jnp.float32)
        y = y + b_ref[...]
        o_ref[...] = jnp.where(y >= 0, y, 0.01 * y)

    return pl.pallas_call(
        body,
        grid=(n // blk,),
        in_specs=[
            pl.BlockSpec((blk, d_in), lambda i: (i, 0)),
            pl.BlockSpec((d_in, d_out), lambda i: (0, 0)),
            pl.BlockSpec((1, d_out), lambda i: (0, 0)),
        ],
        out_specs=pl.BlockSpec((blk, d_out), lambda i: (i, 0)),
        out_shape=jax.ShapeDtypeStruct((n, d_out), jnp.float32),
        compiler_params=pltpu.CompilerParams(
            dimension_semantics=("arbitrary",),
        ),
    )(features, fc_wT, fc_b2d)


def _seg_max_concat(h, idx_padded, n, d, num_segments, nw, chunk):
    p = n // nw

    mesh = plsc.VectorSubcoreMesh(core_axis_name="c", subcore_axis_name="s")

    @functools.partial(
        pl.kernel,
        out_type=(
            jax.ShapeDtypeStruct((n, d), jnp.float32),
            jax.ShapeDtypeStruct((num_segments, d), jnp.float32),
        ),
        mesh=mesh,
        compiler_params=pltpu.CompilerParams(needs_layout_passes=False,
                                             use_tc_tiling_on_sc=False),
        scratch_types=[
            pltpu.VMEM((2 * chunk, d), jnp.float32),
            pltpu.VMEM((2 * (chunk + 8) + 16,), jnp.int32),
            pltpu.VMEM((2 * chunk, d), jnp.float32),
            pltpu.VMEM((_AW, d), jnp.float32),
            pltpu.VMEM((64, d), jnp.float32),
            pltpu.SemaphoreType.DMA,
            pltpu.SemaphoreType.DMA,
        ],
    )
    def body(h_hbm, idx_hbm, out_hbm, agg_hbm, hbuf, ibuf, obuf, awbuf, rep,
             hsem, osem):
        info = plsc.get_sparse_core_info()
        nc = info.num_cores
        wid = _worker_id(nc)
        _worker_body(wid, n, d, num_segments, nw, p, chunk,
                     h_hbm, idx_hbm, out_hbm, agg_hbm,
                     hbuf, ibuf, obuf, awbuf, rep, hsem, osem)

    return body(h, idx_padded)


def _worker_body(wid, n, d, num_segments, nw, p, chunk,
                 h_hbm, idx_hbm, out_hbm, agg_hbm, hbuf, ibuf, obuf, awbuf, rep,
                 hsem, osem):
    nd = d // _LANES
    rw0 = wid * p
    rw1 = rw0 + p
    zvec = jnp.zeros((_LANES,), jnp.float32)
    big = jnp.int32(n + 2 * chunk)

    def memset_aw():
        def zrow(i, _):
            for j in range(nd):
                awbuf[i, pl.ds(_LANES * j, _LANES)] = zvec
            return 0

        lax.fori_loop(0, _AW, zrow, 0)

    memset_aw()

    @pl.when(wid > 0)
    def _():
        pltpu.sync_copy(
            idx_hbm.at[pl.ds(pl.multiple_of(rw0 - _LANES, 8), _LANES)],
            ibuf.at[pl.ds(0, _LANES)])

    lo = jnp.where(wid > 0, ibuf[pl.ds(0, _LANES)][_LANES - 1] + 1, 0)
    pltpu.sync_copy(
        idx_hbm.at[pl.ds(pl.multiple_of(rw1 - _LANES, 8), _LANES)],
        ibuf.at[pl.ds(0, _LANES)])
    hi = jnp.where(wid < nw - 1, ibuf[pl.ds(0, _LANES)][_LANES - 1] + 1,
                   num_segments)

    def blk_write(copy_fn, start, count, maxblk):
        nfull = count // maxblk

        def df(i, _):
            copy_fn(start + i * maxblk, maxblk)
            return 0

        lax.fori_loop(0, nfull, df, 0)
        sz = maxblk // 2
        while sz >= 1:
            pos = start + count - (count % (2 * sz))

            def mk(pos=pos, sz=sz):
                @pl.when((count & sz) != 0)
                def _():
                    copy_fn(pos, sz)

            mk()
            sz //= 2

    def aw_advance(cur, aw_lo):
        def cond(w):
            return cur >= w + _AW

        def adv(w):
            pltpu.sync_copy(awbuf, agg_hbm.at[pl.ds(w, _AW), :])
            memset_aw()
            return w + _AW

        return lax.while_loop(cond, adv, aw_lo)

    islot = chunk + 8

    def parity(r):
        return lax.rem((r - rw0) // chunk, 2)

    def issue_load(r):
        par = parity(r)
        r_al = pl.multiple_of(r - lax.rem(r, 8), 8)
        pltpu.async_copy(h_hbm.at[pl.ds(r, chunk), :],
                         hbuf.at[pl.ds(par * chunk, chunk), :], hsem)
        pltpu.async_copy(idx_hbm.at[pl.ds(r_al, islot)],
                         ibuf.at[pl.ds(par * islot, islot)], hsem)

    def wait_load():
        pltpu.make_async_copy(h_hbm.at[pl.ds(0, chunk), :],
                              hbuf.at[pl.ds(0, chunk), :], hsem).wait()
        pltpu.make_async_copy(idx_hbm.at[pl.ds(0, islot)],
                              ibuf.at[pl.ds(0, islot)], hsem).wait()

    def wait_out():
        pltpu.make_async_copy(h_hbm.at[pl.ds(0, chunk), :],
                              obuf.at[pl.ds(0, chunk), :], osem).wait()

    def rep_patch(a, cnt, accs):
        fill = jnp.minimum(cnt, 64)

        def frow(i, _):
            for j in range(nd):
                rep[i, pl.ds(_LANES * j, _LANES)] = accs[j]
            return 0

        lax.fori_loop(0, fill, frow, 0)

        def pcopy(pos, sz):
            pltpu.sync_copy(rep.at[pl.ds(0, sz), :],
                            out_hbm.at[pl.ds(pos, sz), :])

        blk_write(pcopy, a, cnt, 64)

    def flush_in_chunk(r, obase, cur, a, b, aw_lo, opend, accs):
        aw_lo = aw_advance(cur, aw_lo)
        for j in range(nd):
            awbuf[cur - aw_lo, pl.ds(_LANES * j, _LANES)] = accs[j]
        la = jnp.maximum(a, r)

        def srow(k, _):
            for j in range(nd):
                obuf[obase + k, pl.ds(_LANES * j, _LANES)] = accs[j]
            return 0

        lax.fori_loop(la - r, b - r, srow, 0)

        @pl.when((a < r) & (opend > 0))
        def _():
            wait_out()

        @pl.when(a < r)
        def _():
            rep_patch(a, r - a, accs)

        return aw_lo, jnp.where(a < r, 0, opend)

    neg = jnp.full((_LANES,), -3.0e38, jnp.float32)

    def process_chunk(r, st):
        par = parity(r)
        hbase = par * chunk
        ibase = par * islot
        r_al = pl.multiple_of(r - lax.rem(r, 8), 8)
        off = r - r_al
        wait_load()

        @pl.when(r + chunk < n)
        def _():
            issue_load(r + chunk)

        def row_body(k, st):
            s = ibuf[pl.ds(ibase + off + k, _LANES)][0]
            rowv = tuple(hbuf[hbase + k, pl.ds(_LANES * j, _LANES)]
                         for j in range(nd))

            def active_fn(st):
                cur, a, b, fa, aw_lo, opend = st[:6]
                accs = st[6:]

                def same_fn(_):
                    naccs = tuple(jnp.maximum(accs[j], rowv[j])
                                  for j in range(nd))
                    return (cur, a, r + k + 1, fa, aw_lo, opend) + naccs

                def diff_fn(_):
                    naw, nop = lax.cond(
                        cur >= 0,
                        lambda _: flush_in_chunk(r, hbase, cur, a, b,
                                                 aw_lo, opend, accs),
                        lambda _: (aw_lo, opend), 0)
                    nfa = jnp.where(fa < 0, r + k, fa)
                    return (s, r + k, r + k + 1, nfa, naw, nop) + rowv

                return lax.cond(s == cur, same_fn, diff_fn, 0)

            return lax.cond((s >= lo) & (s < hi), active_fn,
                            lambda st: st, st)

        st = lax.fori_loop(0, chunk, row_body, st)

        cur, a, b, fa, aw_lo, opend = st[:6]
        astart = jnp.maximum(r, jnp.where(fa < 0, big, fa))
        aend = jnp.minimum(b, r + chunk)
        cnt = aend - astart
        is_full = (astart == r) & (aend == r + chunk)

        @pl.when(is_full)
        def _():
            @pl.when(opend > 0)
            def _():
                wait_out()

            pltpu.sync_copy(obuf.at[pl.ds(hbase, chunk), :],
                            out_hbm.at[pl.ds(r, chunk), :])

        @pl.when(jnp.logical_not(is_full) & (cnt > 0))
        def _():
            @pl.when(opend > 0)
            def _():
                wait_out()

            def ocopy(pos, sz):
                pltpu.sync_copy(obuf.at[pl.ds(hbase + pos - r, sz), :],
                                out_hbm.at[pl.ds(pos, sz), :])

            blk_write(ocopy, astart, cnt, min(128, chunk))

        nop = jnp.where(cnt > 0, 0, opend).astype(jnp.int32)
        return (cur, a, b, fa, aw_lo, nop) + st[6:]

    issue_load(rw0)
    st0 = (jnp.int32(-1), rw0, rw0, jnp.int32(-1), lo,
           jnp.int32(0)) + tuple(neg for _ in range(nd))

    def main_chunk(i, st):
        return process_chunk(rw0 + i * chunk, st)

    st = lax.fori_loop(0, p // chunk, main_chunk, st0)

    def over_cond(carry):
        r, stopped = carry[0], carry[1]
        return (r < n) & jnp.logical_not(stopped)

    def over_body(carry):
        r = carry[0]
        st = carry[2:]
        r_al = pl.multiple_of(r - lax.rem(r, 8), 8)
        off = r - r_al
        ibase = parity(r) * islot
        st = process_chunk(r, st)
        last = ibuf[pl.ds(ibase + off + chunk - _LANES, _LANES)][_LANES - 1]
        return (r + chunk, last >= hi) + st

    carry = (rw1, jnp.bool_(False)) + st
    carry = lax.while_loop(over_cond, over_body, carry)
    r_exit = carry[0]
    st = carry[2:]

    cur, a, b, fa, aw_lo, opend = st[:6]
    accs = st[6:]

    @pl.when(opend > 0)
    def _():
        wait_out()

    @pl.when(r_exit < n)
    def _():
        wait_load()

    def final_flush(aw_lo):
        naw = aw_advance(cur, aw_lo)
        for j in range(nd):
            awbuf[cur - naw, pl.ds(_LANES * j, _LANES)] = accs[j]
        rep_patch(a, b - a, accs)
        return naw

    aw_lo = lax.cond(cur >= 0, final_flush, lambda w: w, aw_lo)

    def tail_cond(w):
        return w + _AW <= hi

    def tail_adv(w):
        pltpu.sync_copy(awbuf, agg_hbm.at[pl.ds(w, _AW), :])
        memset_aw()
        return w + _AW

    aw_lo = lax.while_loop(tail_cond, tail_adv, aw_lo)

    def awcopy(pos, sz):
        pltpu.sync_copy(awbuf.at[pl.ds(pos - aw_lo, sz), :],
                        agg_hbm.at[pl.ds(pos, sz), :])

    blk_write(awcopy, aw_lo, hi - aw_lo, _AW // 2)


def kernel(features, agg_indices, fc_w, fc_b):
    n, d_in = features.shape
    d_out = fc_w.shape[0]

    h = _linear_leaky(features, fc_w.T, fc_b.reshape(1, d_out), blk=2000)

    idx32 = agg_indices.astype(jnp.int32)
    idx_padded = jnp.concatenate(
        [idx32, jnp.full((_LANES,), _NUM_SEGMENTS, jnp.int32)])

    out_right, agg = _seg_max_concat(h, idx_padded, n, d_out, _NUM_SEGMENTS,
                                     nw=32, chunk=200)
    out = jnp.concatenate([h, out_right], axis=-1)
    return out, agg


# speedup vs baseline: 2.9769x; 1.1327x over previous
import functools

import jax
import jax.numpy as jnp
from jax import lax
from jax.experimental import pallas as pl
from jax.experimental.pallas import tpu as pltpu
from jax.experimental.pallas import tpu_sc as plsc

_NUM_SEGMENTS = 10000
_LANES = 16
_AW = 64


def _worker_id(nc):
    return lax.axis_index("s") * nc + lax.axis_index("c")


def _linear_leaky(features, fc_wT, fc_b2d, blk):
    n, d_in = features.shape
    d_out = fc_wT.shape[1]

    def body(x_ref, w_ref, b_ref, o_ref):
        y = jnp.dot(x_ref[...], w_ref[...], preferred_element_type=jnp.float32)
        y = y + b_ref[...]
        o_ref[...] = jnp.where(y >= 0, y, 0.01 * y)

    return pl.pallas_call(
        body,
        grid=(n // blk,),
        in_specs=[
            pl.BlockSpec((blk, d_in), lambda i: (i, 0)),
            pl.BlockSpec((d_in, d_out), lambda i: (0, 0)),
            pl.BlockSpec((1, d_out), lambda i: (0, 0)),
        ],
        out_specs=pl.BlockSpec((blk, d_out), lambda i: (i, 0)),
        out_shape=jax.ShapeDtypeStruct((n, d_out), jnp.float32),
        compiler_params=pltpu.CompilerParams(
            dimension_semantics=("arbitrary",),
        ),
    )(features, fc_wT, fc_b2d)


def _seg_max_concat(h, idx_padded, n, d, num_segments, nw, chunk):
    p = n // nw

    mesh = plsc.VectorSubcoreMesh(core_axis_name="c", subcore_axis_name="s")

    @functools.partial(
        pl.kernel,
        out_type=(
            jax.ShapeDtypeStruct((n, d), jnp.float32),
            jax.ShapeDtypeStruct((num_segments, d), jnp.float32),
        ),
        mesh=mesh,
        compiler_params=pltpu.CompilerParams(needs_layout_passes=False,
                                             use_tc_tiling_on_sc=False),
        scratch_types=[
            pltpu.VMEM((2 * chunk, d), jnp.float32),
            pltpu.VMEM((2 * (chunk + 8) + 16,), jnp.int32),
            pltpu.VMEM((2 * chunk, d), jnp.float32),
            pltpu.VMEM((_AW, d), jnp.float32),
            pltpu.VMEM((64, d), jnp.float32),
            pltpu.SemaphoreType.DMA,
            pltpu.SemaphoreType.DMA,
        ],
    )
    def body(h_hbm, idx_hbm, out_hbm, agg_hbm, hbuf, ibuf, obuf, awbuf, rep,
             hsem, osem):
        info = plsc.get_sparse_core_info()
        nc = info.num_cores
        wid = _worker_id(nc)
        _worker_body(wid, n, d, num_segments, nw, p, chunk,
                     h_hbm, idx_hbm, out_hbm, agg_hbm,
                     hbuf, ibuf, obuf, awbuf, rep, hsem, osem)

    return body(h, idx_padded)


def _worker_body(wid, n, d, num_segments, nw, p, chunk,
                 h_hbm, idx_hbm, out_hbm, agg_hbm, hbuf, ibuf, obuf, awbuf, rep,
                 hsem, osem):
    nd = d // _LANES
    rw0 = wid * p
    rw1 = rw0 + p
    zvec = jnp.zeros((_LANES,), jnp.float32)
    big = jnp.int32(n + 2 * chunk)

    def memset_aw():
        def zrow(i, _):
            for j in range(nd):
                awbuf[i, pl.ds(_LANES * j, _LANES)] = zvec
            return 0

        lax.fori_loop(0, _AW, zrow, 0)

    memset_aw()

    @pl.when(wid > 0)
    def _():
        pltpu.sync_copy(
            idx_hbm.at[pl.ds(pl.multiple_of(rw0 - _LANES, 8), _LANES)],
            ibuf.at[pl.ds(0, _LANES)])

    lo = jnp.where(wid > 0, ibuf[pl.ds(0, _LANES)][_LANES - 1] + 1, 0)
    pltpu.sync_copy(
        idx_hbm.at[pl.ds(pl.multiple_of(rw1 - _LANES, 8), _LANES)],
        ibuf.at[pl.ds(0, _LANES)])
    hi = jnp.where(wid < nw - 1, ibuf[pl.ds(0, _LANES)][_LANES - 1] + 1,
                   num_segments)

    def blk_write(copy_fn, start, count, maxblk):
        nfull = count // maxblk

        def df(i, _):
            copy_fn(start + i * maxblk, maxblk)
            return 0

        lax.fori_loop(0, nfull, df, 0)
        sz = maxblk // 2
        while sz >= 1:
            pos = start + count - (count % (2 * sz))

            def mk(pos=pos, sz=sz):
                @pl.when((count & sz) != 0)
                def _():
                    copy_fn(pos, sz)

            mk()
            sz //= 2

    def aw_advance(cur, aw_lo):
        def cond(w):
            return cur >= w + _AW

        def adv(w):
            pltpu.sync_copy(awbuf, agg_hbm.at[pl.ds(w, _AW), :])
            memset_aw()
            return w + _AW

        return lax.while_loop(cond, adv, aw_lo)

    islot = chunk + 8

    def parity(r):
        return lax.rem((r - rw0) // chunk, 2)

    def issue_load(r):
        par = parity(r)
        r_al = pl.multiple_of(r - lax.rem(r, 8), 8)
        pltpu.async_copy(h_hbm.at[pl.ds(r, chunk), :],
                         hbuf.at[pl.ds(par * chunk, chunk), :], hsem)
        pltpu.async_copy(idx_hbm.at[pl.ds(r_al, islot)],
                         ibuf.at[pl.ds(par * islot, islot)], hsem)

    def wait_load():
        pltpu.make_async_copy(h_hbm.at[pl.ds(0, chunk), :],
                              hbuf.at[pl.ds(0, chunk), :], hsem).wait()
        pltpu.make_async_copy(idx_hbm.at[pl.ds(0, islot)],
                              ibuf.at[pl.ds(0, islot)], hsem).wait()

    def wait_out():
        pltpu.make_async_copy(h_hbm.at[pl.ds(0, chunk), :],
                              obuf.at[pl.ds(0, chunk), :], osem).wait()

    def rep_patch(a, cnt, accs):
        fill = jnp.minimum(cnt, 64)

        def frow(i, _):
            for j in range(nd):
                rep[i, pl.ds(_LANES * j, _LANES)] = accs[j]
            return 0

        lax.fori_loop(0, fill, frow, 0)

        def pcopy(pos, sz):
            pltpu.sync_copy(rep.at[pl.ds(0, sz), :],
                            out_hbm.at[pl.ds(pos, sz), :])

        blk_write(pcopy, a, cnt, 64)

    def flush_in_chunk(r, obase, cur, a, b, aw_lo, opend, accs):
        aw_lo = aw_advance(cur, aw_lo)
        for j in range(nd):
            awbuf[cur - aw_lo, pl.ds(_LANES * j, _LANES)] = accs[j]
        la = jnp.maximum(a, r)

        def srow(k, _):
            for j in range(nd):
                obuf[obase + k, pl.ds(_LANES * j, _LANES)] = accs[j]
            return 0

        lax.fori_loop(la - r, b - r, srow, 0)

        @pl.when((a < r) & (opend > 0))
        def _():
            wait_out()

        @pl.when(a < r)
        def _():
            rep_patch(a, r - a, accs)

        return aw_lo, jnp.where(a < r, 0, opend)

    neg = jnp.full((_LANES,), -3.0e38, jnp.float32)

    def process_chunk(r, st):
        par = parity(r)
        hbase = par * chunk
        ibase = par * islot
        r_al = pl.multiple_of(r - lax.rem(r, 8), 8)
        off = r - r_al
        wait_load()

        @pl.when(r + chunk < n)
        def _():
            issue_load(r + chunk)

        def row_body(k, st):
            s = ibuf[pl.ds(ibase + off + k, _LANES)][0]
            rowv = tuple(hbuf[hbase + k, pl.ds(_LANES * j, _LANES)]
                         for j in range(nd))

            def active_fn(st):
                cur, a, b, fa, aw_lo, opend = st[:6]
                accs = st[6:]

                def same_fn(_):
                    naccs = tuple(jnp.maximum(accs[j], rowv[j])
                                  for j in range(nd))
                    return (cur, a, r + k + 1, fa, aw_lo, opend) + naccs

                def diff_fn(_):
                    naw, nop = lax.cond(
                        cur >= 0,
                        lambda _: flush_in_chunk(r, hbase, cur, a, b,
                                                 aw_lo, opend, accs),
                        lambda _: (aw_lo, opend), 0)
                    nfa = jnp.where(fa < 0, r + k, fa)
                    return (s, r + k, r + k + 1, nfa, naw, nop) + rowv

                return lax.cond(s == cur, same_fn, diff_fn, 0)

            return lax.cond((s >= lo) & (s < hi), active_fn,
                            lambda st: st, st)

        ngroups = chunk // _LANES

        def group_body(g, st):
            cur = st[0]
            g16 = g * _LANES
            i16 = ibuf[pl.ds(ibase + off + g16, _LANES)]
            uniform = (jnp.min(i16) == cur) & (jnp.max(i16) == cur)

            def fast(st):
                cur, a, b, fa, aw_lo, opend = st[:6]
                accs = st[6:]
                naccs = []
                for j in range(nd):
                    vals = [hbuf[hbase + g16 + t, pl.ds(_LANES * j, _LANES)]
                            for t in range(_LANES)]
                    while len(vals) > 1:
                        vals = [jnp.maximum(vals[2 * i], vals[2 * i + 1])
                                for i in range(len(vals) // 2)]
                    naccs.append(jnp.maximum(accs[j], vals[0]))
                return (cur, a, r + g16 + _LANES, fa, aw_lo, opend) + tuple(naccs)

            def slow(st):
                return lax.fori_loop(g16, g16 + _LANES, row_body, st)

            return lax.cond(uniform, fast, slow, st)

        st = lax.fori_loop(0, ngroups, group_body, st)
        st = lax.fori_loop(ngroups * _LANES, chunk, row_body, st)

        cur, a, b, fa, aw_lo, opend = st[:6]
        astart = jnp.maximum(r, jnp.where(fa < 0, big, fa))
        aend = jnp.minimum(b, r + chunk)
        cnt = aend - astart
        is_full = (astart == r) & (aend == r + chunk)

        @pl.when(is_full)
        def _():
            @pl.when(opend > 0)
            def _():
                wait_out()

            pltpu.sync_copy(obuf.at[pl.ds(hbase, chunk), :],
                            out_hbm.at[pl.ds(r, chunk), :])

        @pl.when(jnp.logical_not(is_full) & (cnt > 0))
        def _():
            @pl.when(opend > 0)
            def _():
                wait_out()

            def ocopy(pos, sz):
                pltpu.sync_copy(obuf.at[pl.ds(hbase + pos - r, sz), :],
                                out_hbm.at[pl.ds(pos, sz), :])

            blk_write(ocopy, astart, cnt, min(128, chunk))

        nop = jnp.where(cnt > 0, 0, opend).astype(jnp.int32)
        return (cur, a, b, fa, aw_lo, nop) + st[6:]

    issue_load(rw0)
    st0 = (jnp.int32(-1), rw0, rw0, jnp.int32(-1), lo,
           jnp.int32(0)) + tuple(neg for _ in range(nd))

    def main_chunk(i, st):
        return process_chunk(rw0 + i * chunk, st)

    st = lax.fori_loop(0, p // chunk, main_chunk, st0)

    def over_cond(carry):
        r, stopped = carry[0], carry[1]
        return (r < n) & jnp.logical_not(stopped)

    def over_body(carry):
        r = carry[0]
        st = carry[2:]
        r_al = pl.multiple_of(r - lax.rem(r, 8), 8)
        off = r - r_al
        ibase = parity(r) * islot
        st = process_chunk(r, st)
        last = ibuf[pl.ds(ibase + off + chunk - _LANES, _LANES)][_LANES - 1]
        return (r + chunk, last >= hi) + st

    carry = (rw1, jnp.bool_(False)) + st
    carry = lax.while_loop(over_cond, over_body, carry)
    r_exit = carry[0]
    st = carry[2:]

    cur, a, b, fa, aw_lo, opend = st[:6]
    accs = st[6:]

    @pl.when(opend > 0)
    def _():
        wait_out()

    @pl.when(r_exit < n)
    def _():
        wait_load()

    def final_flush(aw_lo):
        naw = aw_advance(cur, aw_lo)
        for j in range(nd):
            awbuf[cur - naw, pl.ds(_LANES * j, _LANES)] = accs[j]
        rep_patch(a, b - a, accs)
        return naw

    aw_lo = lax.cond(cur >= 0, final_flush, lambda w: w, aw_lo)

    def tail_cond(w):
        return w + _AW <= hi

    def tail_adv(w):
        pltpu.sync_copy(awbuf, agg_hbm.at[pl.ds(w, _AW), :])
        memset_aw()
        return w + _AW

    aw_lo = lax.while_loop(tail_cond, tail_adv, aw_lo)

    def awcopy(pos, sz):
        pltpu.sync_copy(awbuf.at[pl.ds(pos - aw_lo, sz), :],
                        agg_hbm.at[pl.ds(pos, sz), :])

    blk_write(awcopy, aw_lo, hi - aw_lo, _AW // 2)


def kernel(features, agg_indices, fc_w, fc_b):
    n, d_in = features.shape
    d_out = fc_w.shape[0]

    h = _linear_leaky(features, fc_w.T, fc_b.reshape(1, d_out), blk=2000)

    idx32 = agg_indices.astype(jnp.int32)
    idx_padded = jnp.concatenate(
        [idx32, jnp.full((_LANES,), _NUM_SEGMENTS, jnp.int32)])

    out_right, agg = _seg_max_concat(h, idx_padded, n, d_out, _NUM_SEGMENTS,
                                     nw=32, chunk=200)
    out = jnp.concatenate([h, out_right], axis=-1)
    return out, agg


# speedup vs baseline: 2.9874x; 1.0035x over previous
import functools

import jax
import jax.numpy as jnp
from jax import lax
from jax.experimental import pallas as pl
from jax.experimental.pallas import tpu as pltpu
from jax.experimental.pallas import tpu_sc as plsc

_NUM_SEGMENTS = 10000
_LANES = 16
_AW = 64


def _worker_id(nc):
    return lax.axis_index("s") * nc + lax.axis_index("c")


def _linear_leaky(features, fc_wT, fc_b2d, blk):
    n, d_in = features.shape
    d_out = fc_wT.shape[1]

    def body(x_ref, w_ref, b_ref, o_ref):
        y = jnp.dot(x_ref[...], w_ref[...], preferred_element_type=jnp.float32)
        y = y + b_ref[...]
        o_ref[...] = jnp.where(y >= 0, y, 0.01 * y)

    return pl.pallas_call(
        body,
        grid=(n // blk,),
        in_specs=[
            pl.BlockSpec((blk, d_in), lambda i: (i, 0)),
            pl.BlockSpec((d_in, d_out), lambda i: (0, 0)),
            pl.BlockSpec((1, d_out), lambda i: (0, 0)),
        ],
        out_specs=pl.BlockSpec((blk, d_out), lambda i: (i, 0)),
        out_shape=jax.ShapeDtypeStruct((n, d_out), jnp.float32),
        compiler_params=pltpu.CompilerParams(
            dimension_semantics=("arbitrary",),
        ),
    )(features, fc_wT, fc_b2d)


def _seg_max_concat(h, idx_padded, n, d, num_segments, nw, chunk):
    p = n // nw

    mesh = plsc.VectorSubcoreMesh(core_axis_name="c", subcore_axis_name="s")

    @functools.partial(
        pl.kernel,
        out_type=(
            jax.ShapeDtypeStruct((n, d), jnp.float32),
            jax.ShapeDtypeStruct((num_segments, d), jnp.float32),
        ),
        mesh=mesh,
        compiler_params=pltpu.CompilerParams(needs_layout_passes=False,
                                             use_tc_tiling_on_sc=False),
        scratch_types=[
            pltpu.VMEM((2 * chunk, d), jnp.float32),
            pltpu.VMEM((2 * (((chunk + 15) // 8) * 8) + 16,), jnp.int32),
            pltpu.VMEM((chunk, d), jnp.float32),
            pltpu.VMEM((_AW, d), jnp.float32),
            pltpu.VMEM((64, d), jnp.float32),
            pltpu.SemaphoreType.DMA,
            pltpu.SemaphoreType.DMA,
        ],
    )
    def body(h_hbm, idx_hbm, out_hbm, agg_hbm, hbuf, ibuf, obuf, awbuf, rep,
             hsem, osem):
        info = plsc.get_sparse_core_info()
        nc = info.num_cores
        wid = _worker_id(nc)
        _worker_body(wid, n, d, num_segments, nw, p, chunk,
                     h_hbm, idx_hbm, out_hbm, agg_hbm,
                     hbuf, ibuf, obuf, awbuf, rep, hsem, osem)

    return body(h, idx_padded)


def _worker_body(wid, n, d, num_segments, nw, p, chunk,
                 h_hbm, idx_hbm, out_hbm, agg_hbm, hbuf, ibuf, obuf, awbuf, rep,
                 hsem, osem):
    nd = d // _LANES
    rw0 = wid * p
    rw1 = rw0 + p
    zvec = jnp.zeros((_LANES,), jnp.float32)
    big = jnp.int32(n + 2 * chunk)

    def memset_aw():
        def zrow(i, _):
            for j in range(nd):
                awbuf[i, pl.ds(_LANES * j, _LANES)] = zvec
            return 0

        lax.fori_loop(0, _AW, zrow, 0)

    memset_aw()

    @pl.when(wid > 0)
    def _():
        pltpu.sync_copy(
            idx_hbm.at[pl.ds(pl.multiple_of(rw0 - _LANES, 8), _LANES)],
            ibuf.at[pl.ds(0, _LANES)])

    lo = jnp.where(wid > 0, ibuf[pl.ds(0, _LANES)][_LANES - 1] + 1, 0)
    pltpu.sync_copy(
        idx_hbm.at[pl.ds(pl.multiple_of(rw1 - _LANES, 8), _LANES)],
        ibuf.at[pl.ds(0, _LANES)])
    hi = jnp.where(wid < nw - 1, ibuf[pl.ds(0, _LANES)][_LANES - 1] + 1,
                   num_segments)

    def blk_write(copy_fn, start, count, maxblk):
        nfull = count // maxblk

        def df(i, _):
            copy_fn(start + i * maxblk, maxblk)
            return 0

        lax.fori_loop(0, nfull, df, 0)
        sz = maxblk // 2
        while sz >= 1:
            pos = start + count - (count % (2 * sz))

            def mk(pos=pos, sz=sz):
                @pl.when((count & sz) != 0)
                def _():
                    copy_fn(pos, sz)

            mk()
            sz //= 2

    def aw_advance(cur, aw_lo):
        def cond(w):
            return cur >= w + _AW

        def adv(w):
            pltpu.sync_copy(awbuf, agg_hbm.at[pl.ds(w, _AW), :])
            memset_aw()
            return w + _AW

        return lax.while_loop(cond, adv, aw_lo)

    islot = ((chunk + 15) // 8) * 8

    def parity(r):
        return lax.rem((r - rw0) // chunk, 2)

    def issue_load(r):
        par = parity(r)
        r_al = pl.multiple_of(r - lax.rem(r, 8), 8)
        pltpu.async_copy(h_hbm.at[pl.ds(r, chunk), :],
                         hbuf.at[pl.ds(par * chunk, chunk), :], hsem)
        pltpu.async_copy(idx_hbm.at[pl.ds(r_al, islot)],
                         ibuf.at[pl.ds(par * islot, islot)], hsem)

    def wait_load():
        pltpu.make_async_copy(h_hbm.at[pl.ds(0, chunk), :],
                              hbuf.at[pl.ds(0, chunk), :], hsem).wait()
        pltpu.make_async_copy(idx_hbm.at[pl.ds(0, islot)],
                              ibuf.at[pl.ds(0, islot)], hsem).wait()

    def wait_out():
        pltpu.make_async_copy(h_hbm.at[pl.ds(0, chunk), :],
                              obuf.at[pl.ds(0, chunk), :], osem).wait()

    def rep_patch(a, cnt, accs):
        fill = jnp.minimum(cnt, 64)

        def frow(i, _):
            for j in range(nd):
                rep[i, pl.ds(_LANES * j, _LANES)] = accs[j]
            return 0

        lax.fori_loop(0, fill, frow, 0)

        def pcopy(pos, sz):
            pltpu.sync_copy(rep.at[pl.ds(0, sz), :],
                            out_hbm.at[pl.ds(pos, sz), :])

        blk_write(pcopy, a, cnt, 64)

    def flush_in_chunk(r, obase, cur, a, b, aw_lo, opend, accs):
        aw_lo = aw_advance(cur, aw_lo)
        for j in range(nd):
            awbuf[cur - aw_lo, pl.ds(_LANES * j, _LANES)] = accs[j]
        la = jnp.maximum(a, r)

        def srow(k, _):
            for j in range(nd):
                obuf[obase + k, pl.ds(_LANES * j, _LANES)] = accs[j]
            return 0

        lax.fori_loop(la - r, b - r, srow, 0)

        @pl.when((a < r) & (opend > 0))
        def _():
            wait_out()

        @pl.when(a < r)
        def _():
            rep_patch(a, r - a, accs)

        return aw_lo, jnp.where(a < r, 0, opend)

    neg = jnp.full((_LANES,), -3.0e38, jnp.float32)

    def process_chunk(r, st):
        par = parity(r)
        hbase = par * chunk
        ibase = par * islot
        r_al = pl.multiple_of(r - lax.rem(r, 8), 8)
        off = r - r_al
        wait_load()

        @pl.when(r + chunk < n)
        def _():
            issue_load(r + chunk)

        def row_body(k, st):
            s = ibuf[pl.ds(ibase + off + k, _LANES)][0]
            rowv = tuple(hbuf[hbase + k, pl.ds(_LANES * j, _LANES)]
                         for j in range(nd))

            def active_fn(st):
                cur, a, b, fa, aw_lo, opend = st[:6]
                accs = st[6:]

                def same_fn(_):
                    naccs = tuple(jnp.maximum(accs[j], rowv[j])
                                  for j in range(nd))
                    return (cur, a, r + k + 1, fa, aw_lo, opend) + naccs

                def diff_fn(_):
                    naw, nop = lax.cond(
                        cur >= 0,
                        lambda _: flush_in_chunk(r, 0, cur, a, b,
                                                 aw_lo, opend, accs),
                        lambda _: (aw_lo, opend), 0)
                    nfa = jnp.where(fa < 0, r + k, fa)
                    return (s, r + k, r + k + 1, nfa, naw, nop) + rowv

                return lax.cond(s == cur, same_fn, diff_fn, 0)

            return lax.cond((s >= lo) & (s < hi), active_fn,
                            lambda st: st, st)

        ngroups = chunk // _LANES

        def group_body(g, st):
            cur = st[0]
            g16 = g * _LANES
            i16 = ibuf[pl.ds(ibase + off + g16, _LANES)]
            uniform = (jnp.min(i16) == cur) & (jnp.max(i16) == cur)

            def fast(st):
                cur, a, b, fa, aw_lo, opend = st[:6]
                accs = st[6:]
                naccs = []
                for j in range(nd):
                    vals = [hbuf[hbase + g16 + t, pl.ds(_LANES * j, _LANES)]
                            for t in range(_LANES)]
                    while len(vals) > 1:
                        vals = [jnp.maximum(vals[2 * i], vals[2 * i + 1])
                                for i in range(len(vals) // 2)]
                    naccs.append(jnp.maximum(accs[j], vals[0]))
                return (cur, a, r + g16 + _LANES, fa, aw_lo, opend) + tuple(naccs)

            def slow(st):
                return lax.fori_loop(g16, g16 + _LANES, row_body, st)

            return lax.cond(uniform, fast, slow, st)

        st = lax.fori_loop(0, ngroups, group_body, st)
        st = lax.fori_loop(ngroups * _LANES, chunk, row_body, st)

        cur, a, b, fa, aw_lo, opend = st[:6]
        astart = jnp.maximum(r, jnp.where(fa < 0, big, fa))
        aend = jnp.minimum(b, r + chunk)
        cnt = aend - astart
        is_full = (astart == r) & (aend == r + chunk)

        @pl.when(is_full)
        def _():
            @pl.when(opend > 0)
            def _():
                wait_out()

            pltpu.sync_copy(obuf, out_hbm.at[pl.ds(r, chunk), :])

        @pl.when(jnp.logical_not(is_full) & (cnt > 0))
        def _():
            @pl.when(opend > 0)
            def _():
                wait_out()

            def ocopy(pos, sz):
                pltpu.sync_copy(obuf.at[pl.ds(pos - r, sz), :],
                                out_hbm.at[pl.ds(pos, sz), :])

            blk_write(ocopy, astart, cnt, min(128, chunk))

        nop = jnp.where(cnt > 0, 0, opend).astype(jnp.int32)
        return (cur, a, b, fa, aw_lo, nop) + st[6:]

    issue_load(rw0)
    st0 = (jnp.int32(-1), rw0, rw0, jnp.int32(-1), lo,
           jnp.int32(0)) + tuple(neg for _ in range(nd))

    def main_chunk(i, st):
        return process_chunk(rw0 + i * chunk, st)

    st = lax.fori_loop(0, p // chunk, main_chunk, st0)

    def over_cond(carry):
        r, stopped = carry[0], carry[1]
        return (r < n) & jnp.logical_not(stopped)

    def over_body(carry):
        r = carry[0]
        st = carry[2:]
        r_al = pl.multiple_of(r - lax.rem(r, 8), 8)
        off = r - r_al
        ibase = parity(r) * islot
        st = process_chunk(r, st)
        last = ibuf[pl.ds(ibase + off + chunk - _LANES, _LANES)][_LANES - 1]
        return (r + chunk, last >= hi) + st

    carry = (rw1, jnp.bool_(False)) + st
    carry = lax.while_loop(over_cond, over_body, carry)
    r_exit = carry[0]
    st = carry[2:]

    cur, a, b, fa, aw_lo, opend = st[:6]
    accs = st[6:]

    @pl.when(opend > 0)
    def _():
        wait_out()

    @pl.when(r_exit < n)
    def _():
        wait_load()

    def final_flush(aw_lo):
        naw = aw_advance(cur, aw_lo)
        for j in range(nd):
            awbuf[cur - naw, pl.ds(_LANES * j, _LANES)] = accs[j]
        rep_patch(a, b - a, accs)
        return naw

    aw_lo = lax.cond(cur >= 0, final_flush, lambda w: w, aw_lo)

    def tail_cond(w):
        return w + _AW <= hi

    def tail_adv(w):
        pltpu.sync_copy(awbuf, agg_hbm.at[pl.ds(w, _AW), :])
        memset_aw()
        return w + _AW

    aw_lo = lax.while_loop(tail_cond, tail_adv, aw_lo)

    def awcopy(pos, sz):
        pltpu.sync_copy(awbuf.at[pl.ds(pos - aw_lo, sz), :],
                        agg_hbm.at[pl.ds(pos, sz), :])

    blk_write(awcopy, aw_lo, hi - aw_lo, _AW // 2)


def kernel(features, agg_indices, fc_w, fc_b):
    n, d_in = features.shape
    d_out = fc_w.shape[0]

    h = _linear_leaky(features, fc_w.T, fc_b.reshape(1, d_out), blk=2000)

    idx32 = agg_indices.astype(jnp.int32)
    idx_padded = jnp.concatenate(
        [idx32, jnp.full((2 * _LANES,), _NUM_SEGMENTS, jnp.int32)])

    out_right, agg = _seg_max_concat(h, idx_padded, n, d_out, _NUM_SEGMENTS,
                                     nw=32, chunk=250)
    out = jnp.concatenate([h, out_right], axis=-1)
    return out, agg
